# baseline (speedup 1.0000x reference)
import functools

import jax
import jax.numpy as jnp
from jax import lax
from jax.experimental import pallas as pl
from jax.experimental.pallas import tpu as pltpu
from jax.experimental.pallas import tpu_sc as plsc

H = 30
W = 30
C = 64
D = 16
K = 3
B = 1024
N = H * W

BB = 8
G = B // BB
NC = 2
NS = 16
NW = NC * NS
PER = B // NW
L = 16
NP = 1024
NFULL = N // L
R = D + K + 1


def _tc_proj_body(x_ref, wqT_ref, bq_ref, wvT_ref, bv_ref, comb_ref):
    wqT = wqT_ref[...]
    wvT = wvT_ref[...]
    bq = bq_ref[...]
    bv = bv_ref[...]
    for b in range(BB):
        x = x_ref[b]
        logits = jnp.dot(wqT, x, preferred_element_type=jnp.float32) + bq
        m = jnp.max(logits, axis=0, keepdims=True)
        e = jnp.exp(logits - m)
        ssum = jnp.sum(e, axis=0, keepdims=True)
        attn = e / ssum
        vals = jnp.dot(wvT, x, preferred_element_type=jnp.float32) + bv
        comb_ref[pl.ds(b * R, D), pl.ds(0, N)] = vals
        comb_ref[pl.ds(b * R + D, K), pl.ds(0, N)] = attn


def _tc_project(state3, wqT, bq2, wvT, bv2):
    return pl.pallas_call(
        _tc_proj_body,
        grid=(G,),
        in_specs=[
            pl.BlockSpec((BB, C, N), lambda i: (i, 0, 0)),
            pl.BlockSpec((K, C), lambda i: (0, 0)),
            pl.BlockSpec((K, 1), lambda i: (0, 0)),
            pl.BlockSpec((D, C), lambda i: (0, 0)),
            pl.BlockSpec((D, 1), lambda i: (0, 0)),
        ],
        out_specs=pl.BlockSpec((BB * R, NP), lambda i: (i, 0)),
        out_shape=jax.ShapeDtypeStruct((B * R, NP), jnp.float32),
        compiler_params=pltpu.CompilerParams(
            dimension_semantics=("arbitrary",)),
    )(state3, wqT, bq2, wvT, bv2)


def _sc_body(comb_hbm, part_hbm, out_hbm, pbuf, ibuf0, ibuf1, obuf0, obuf1,
             sin0, sin1, sout0, sout1):
    c = lax.axis_index("c")
    s = lax.axis_index("s")
    base = (s * NC + c) * PER
    pltpu.sync_copy(part_hbm, pbuf)

    ibufs = (ibuf0, ibuf1)
    obufs = (obuf0, obuf1)
    sins = (sin0, sin1)
    souts = (sout0, sout1)

    def start_in(par, j):
        pltpu.make_async_copy(comb_hbm.at[base + j], ibufs[par], sins[par]).start()

    def wait_in(par):
        pltpu.make_async_copy(comb_hbm.at[base], ibufs[par], sins[par]).wait()

    def start_out(par, j):
        dst = out_hbm.at[pl.ds((base + j) * D * N, D * N)]
        pltpu.make_async_copy(obufs[par], dst, souts[par]).start()

    def wait_out(par):
        dst = out_hbm.at[pl.ds(base * D * N, D * N)]
        pltpu.make_async_copy(obufs[par], dst, souts[par]).wait()

    def compute(ibuf, obuf):
        @plsc.parallel_loop(0, NFULL * L, L, unroll=1)
        def chunk_body(i0):
            a0 = ibuf[pl.ds(pl.multiple_of(D * NP + i0, L), L)]
            a1 = ibuf[pl.ds(pl.multiple_of((D + 1) * NP + i0, L), L)]
            a2 = ibuf[pl.ds(pl.multiple_of((D + 2) * NP + i0, L), L)]
            p0 = pbuf[pl.ds(pl.multiple_of(i0, L), L)]
            p1 = pbuf[pl.ds(pl.multiple_of(NP + i0, L), L)]
            p2 = pbuf[pl.ds(pl.multiple_of(2 * NP + i0, L), L)]
            pos = i0 + lax.iota(jnp.int32, L)
            for d in range(D):
                off = d * NP
                g0 = plsc.load_gather(ibuf, [p0 + off])
                g1 = plsc.load_gather(ibuf, [p1 + off])
                g2 = plsc.load_gather(ibuf, [p2 + off])
                plsc.store_scatter(obuf, [pos + d * N],
                                   a0 * g0 + a1 * g1 + a2 * g2)

        t0 = NFULL * L
        posv = t0 + lax.iota(jnp.int32, L)
        msk = posv < N
        posc = jnp.minimum(posv, N - 1)
        a0 = ibuf[pl.ds(D * NP + t0, L)]
        a1 = ibuf[pl.ds((D + 1) * NP + t0, L)]
        a2 = ibuf[pl.ds((D + 2) * NP + t0, L)]
        p0 = pbuf[pl.ds(t0, L)]
        p1 = pbuf[pl.ds(NP + t0, L)]
        p2 = pbuf[pl.ds(2 * NP + t0, L)]
        for d in range(D):
            off = d * NP
            g0 = plsc.load_gather(ibuf, [p0 + off])
            g1 = plsc.load_gather(ibuf, [p1 + off])
            g2 = plsc.load_gather(ibuf, [p2 + off])
            plsc.store_scatter(obuf, [posc + d * N],
                               a0 * g0 + a1 * g1 + a2 * g2, mask=msk)

    start_in(0, 0)
    start_in(1, 1)

    def outer(t, carry):
        j0 = t * 2
        for par in range(2):
            j = j0 + par
            wait_in(par)

            @pl.when(j >= 2)
            def _():
                wait_out(par)

            compute(ibufs[par], obufs[par])
            start_out(par, j)

            @pl.when(j + 2 < PER)
            def _():
                start_in(par, j + 2)
        return carry

    lax.fori_loop(0, PER // 2, outer, 0)
    wait_out(0)
    wait_out(1)


def _sc_gather(comb2, partsF):
    mesh = plsc.VectorSubcoreMesh(core_axis_name="c", subcore_axis_name="s")
    run = functools.partial(
        pl.kernel,
        mesh=mesh,
        compiler_params=pltpu.CompilerParams(
            use_tc_tiling_on_sc=False, needs_layout_passes=False),
        out_type=jax.ShapeDtypeStruct((B * D * N,), jnp.float32),
        scratch_types=[
            pltpu.VMEM((K * NP,), jnp.int32),
            pltpu.VMEM((R * NP,), jnp.float32),
            pltpu.VMEM((R * NP,), jnp.float32),
            pltpu.VMEM((D * N,), jnp.float32),
            pltpu.VMEM((D * N,), jnp.float32),
            pltpu.SemaphoreType.DMA,
            pltpu.SemaphoreType.DMA,
            pltpu.SemaphoreType.DMA,
            pltpu.SemaphoreType.DMA,
        ],
    )(_sc_body)
    return run(comb2, partsF)


def kernel(state, partners, Wq, bq, Wv, bv):
    state3 = state.reshape(B, C, N)
    wqT = Wq.T
    wvT = Wv.T
    bq2 = bq.reshape(K, 1)
    bv2 = bv.reshape(D, 1)
    partsF = (jnp.zeros((K, NP), jnp.int32)
              .at[:, :N].set(partners.astype(jnp.int32).T)
              .reshape(K * NP))
    comb = _tc_project(state3, wqT, bq2, wvT, bv2)
    out1 = _sc_gather(comb.reshape(B, R * NP), partsF)
    return out1.reshape(B, D, H, W)


# speedup vs baseline: 1.0269x; 1.0269x over previous
import functools

import jax
import jax.numpy as jnp
from jax import lax
from jax.experimental import pallas as pl
from jax.experimental.pallas import tpu as pltpu
from jax.experimental.pallas import tpu_sc as plsc

H = 30
W = 30
C = 64
D = 16
K = 3
B = 1024
N = H * W

BB = 8
NCH = 4
CB = B // NCH
NC = 2
NS = 16
NW = NC * NS
PER = CB // NW
L = 16
NP = 1024
NFULL = N // L
R = D + K + 1


def _tc_proj_body(x_ref, wqT_ref, bq_ref, wvT_ref, bv_ref, comb_ref):
    wqT = wqT_ref[...]
    wvT = wvT_ref[...]
    bq = bq_ref[...]
    bv = bv_ref[...]
    for b in range(BB):
        x = x_ref[b]
        logits = jnp.dot(wqT, x, preferred_element_type=jnp.float32) + bq
        m = jnp.max(logits, axis=0, keepdims=True)
        e = jnp.exp(logits - m)
        ssum = jnp.sum(e, axis=0, keepdims=True)
        attn = e / ssum
        vals = jnp.dot(wvT, x, preferred_element_type=jnp.float32) + bv
        comb_ref[pl.ds(b * R, D), pl.ds(0, N)] = vals
        comb_ref[pl.ds(b * R + D, K), pl.ds(0, N)] = attn


def _tc_project(state3, wqT, bq2, wvT, bv2, off):
    blk = off // BB
    return pl.pallas_call(
        _tc_proj_body,
        grid=(CB // BB,),
        in_specs=[
            pl.BlockSpec((BB, C, N), lambda i: (i + blk, 0, 0)),
            pl.BlockSpec((K, C), lambda i: (0, 0)),
            pl.BlockSpec((K, 1), lambda i: (0, 0)),
            pl.BlockSpec((D, C), lambda i: (0, 0)),
            pl.BlockSpec((D, 1), lambda i: (0, 0)),
        ],
        out_specs=pl.BlockSpec((BB * R, NP), lambda i: (i, 0)),
        out_shape=jax.ShapeDtypeStruct((CB * R, NP), jnp.float32),
        compiler_params=pltpu.CompilerParams(
            dimension_semantics=("arbitrary",)),
    )(state3, wqT, bq2, wvT, bv2)


def _sc_body(comb_hbm, part_hbm, out_hbm, pbuf, ibuf0, ibuf1, obuf0, obuf1,
             sin0, sin1, sout0, sout1):
    c = lax.axis_index("c")
    s = lax.axis_index("s")
    base = (s * NC + c) * PER
    pltpu.sync_copy(part_hbm, pbuf)

    ibufs = (ibuf0, ibuf1)
    obufs = (obuf0, obuf1)
    sins = (sin0, sin1)
    souts = (sout0, sout1)

    def start_in(par, j):
        pltpu.make_async_copy(comb_hbm.at[base + j], ibufs[par], sins[par]).start()

    def wait_in(par):
        pltpu.make_async_copy(comb_hbm.at[base], ibufs[par], sins[par]).wait()

    def start_out(par, j):
        dst = out_hbm.at[pl.ds((base + j) * D * N, D * N)]
        pltpu.make_async_copy(obufs[par], dst, souts[par]).start()

    def wait_out(par):
        dst = out_hbm.at[pl.ds(base * D * N, D * N)]
        pltpu.make_async_copy(obufs[par], dst, souts[par]).wait()

    def compute(ibuf, obuf):
        @plsc.parallel_loop(0, NFULL * L, L, unroll=1)
        def chunk_body(i0):
            a0 = ibuf[pl.ds(pl.multiple_of(D * NP + i0, L), L)]
            a1 = ibuf[pl.ds(pl.multiple_of((D + 1) * NP + i0, L), L)]
            a2 = ibuf[pl.ds(pl.multiple_of((D + 2) * NP + i0, L), L)]
            p0 = pbuf[pl.ds(pl.multiple_of(i0, L), L)]
            p1 = pbuf[pl.ds(pl.multiple_of(NP + i0, L), L)]
            p2 = pbuf[pl.ds(pl.multiple_of(2 * NP + i0, L), L)]
            pos = i0 + lax.iota(jnp.int32, L)
            for d in range(D):
                off = d * NP
                g0 = plsc.load_gather(ibuf, [p0 + off])
                g1 = plsc.load_gather(ibuf, [p1 + off])
                g2 = plsc.load_gather(ibuf, [p2 + off])
                plsc.store_scatter(obuf, [pos + d * N],
                                   a0 * g0 + a1 * g1 + a2 * g2)

        t0 = NFULL * L
        posv = t0 + lax.iota(jnp.int32, L)
        msk = posv < N
        posc = jnp.minimum(posv, N - 1)
        a0 = ibuf[pl.ds(D * NP + t0, L)]
        a1 = ibuf[pl.ds((D + 1) * NP + t0, L)]
        a2 = ibuf[pl.ds((D + 2) * NP + t0, L)]
        p0 = pbuf[pl.ds(t0, L)]
        p1 = pbuf[pl.ds(NP + t0, L)]
        p2 = pbuf[pl.ds(2 * NP + t0, L)]
        for d in range(D):
            off = d * NP
            g0 = plsc.load_gather(ibuf, [p0 + off])
            g1 = plsc.load_gather(ibuf, [p1 + off])
            g2 = plsc.load_gather(ibuf, [p2 + off])
            plsc.store_scatter(obuf, [posc + d * N],
                               a0 * g0 + a1 * g1 + a2 * g2, mask=msk)

    start_in(0, 0)
    start_in(1, 1)

    def outer(t, carry):
        j0 = t * 2
        for par in range(2):
            j = j0 + par
            wait_in(par)

            @pl.when(j >= 2)
            def _():
                wait_out(par)

            compute(ibufs[par], obufs[par])
            start_out(par, j)

            @pl.when(j + 2 < PER)
            def _():
                start_in(par, j + 2)
        return carry

    lax.fori_loop(0, PER // 2, outer, 0)
    wait_out(0)
    wait_out(1)


def _sc_gather(comb2, partsF):
    mesh = plsc.VectorSubcoreMesh(core_axis_name="c", subcore_axis_name="s")
    run = functools.partial(
        pl.kernel,
        mesh=mesh,
        compiler_params=pltpu.CompilerParams(
            use_tc_tiling_on_sc=False, needs_layout_passes=False),
        out_type=jax.ShapeDtypeStruct((CB * D * N,), jnp.float32),
        scratch_types=[
            pltpu.VMEM((K * NP,), jnp.int32),
            pltpu.VMEM((R * NP,), jnp.float32),
            pltpu.VMEM((R * NP,), jnp.float32),
            pltpu.VMEM((D * N,), jnp.float32),
            pltpu.VMEM((D * N,), jnp.float32),
            pltpu.SemaphoreType.DMA,
            pltpu.SemaphoreType.DMA,
            pltpu.SemaphoreType.DMA,
            pltpu.SemaphoreType.DMA,
        ],
    )(_sc_body)
    return run(comb2, partsF)


def kernel(state, partners, Wq, bq, Wv, bv):
    state3 = state.reshape(B, C, N)
    wqT = Wq.T
    wvT = Wv.T
    bq2 = bq.reshape(K, 1)
    bv2 = bv.reshape(D, 1)
    partsF = (jnp.zeros((K, NP), jnp.int32)
              .at[:, :N].set(partners.astype(jnp.int32).T)
              .reshape(K * NP))
    outs = []
    for ci in range(NCH):
        comb = _tc_project(state3, wqT, bq2, wvT, bv2, ci * CB)
        outs.append(_sc_gather(comb.reshape(CB, R * NP), partsF))
    return jnp.concatenate(outs).reshape(B, D, H, W)
